# baseline (speedup 1.0000x reference)
import jax
import jax.numpy as jnp
from jax import lax
from jax.experimental import pallas as pl
from jax.experimental.pallas import tpu as pltpu
from jax.experimental.pallas import tpu_sc as plsc

SEQ_LEN = 200
DIM = 64
NC = 2
NS = 16
NW = NC * NS
BCH = 128
RW = 128
NIB = 8
NRB = 4
GA = 3
NSB = 2
NPB = 4
RING = 8


def _body(idx_hbm, tab_hbm, posb_hbm, out_hbm,
          idx_bufs, row_bufs, ridx_bufs, par_bufs, slab_bufs, posb_bufs,
          isems, gsems, ssems, psems):
    wid = lax.axis_index("s") * NC + lax.axis_index("c")
    lanes = lax.iota(jnp.int32, 16)

    def idx_fire(l, s):
        pltpu.async_copy(idx_hbm.at[l, pl.ds(wid * BCH, BCH)],
                         idx_bufs[s], isems[s])

    def idx_wait(l, s):
        pltpu.make_async_copy(idx_hbm.at[l, pl.ds(wid * BCH, BCH)],
                              idx_bufs[s], isems[s]).wait()

    def shift_pass(si, sr):
        for k in range(BCH // 16):
            sl = pl.ds(k * 16, 16)
            v = idx_bufs[si][sl]
            ridx_bufs[sr][sl] = lax.shift_right_logical(v, 1)
            par_bufs[sr][sl] = (v & 1) * DIM

    def gather_fire(sr):
        pltpu.async_copy(tab_hbm.at[ridx_bufs[sr]], row_bufs[sr], gsems[sr])

    def gather_wait(sr):
        pltpu.make_async_copy(tab_hbm.at[ridx_bufs[sr]], row_bufs[sr],
                              gsems[sr]).wait()

    def posb_fire(l, s):
        pltpu.async_copy(posb_hbm.at[l], posb_bufs[s], psems[s])

    def posb_wait(l, s):
        pltpu.make_async_copy(posb_hbm.at[l], posb_bufs[s], psems[s]).wait()

    def store_fire(l, s):
        pltpu.async_copy(slab_bufs[s],
                         out_hbm.at[l, :, pl.ds(wid * BCH, BCH)], ssems[s])

    def store_wait(l, s):
        pltpu.make_async_copy(slab_bufs[s],
                              out_hbm.at[l, :, pl.ds(wid * BCH, BCH)],
                              ssems[s]).wait()

    for m in range(NIB):
        idx_fire(m, m)
    for m in range(GA):
        idx_wait(m, m)
        shift_pass(m, m % NRB)
        gather_fire(m % NRB)
    for m in range(NPB):
        posb_fire(m, m)

    def one_round(r, carry):
        for j in range(RING):
            l = r * RING + j
            si, sr, ss, sp = j % NIB, j % NRB, j % NSB, j % NPB

            gather_wait(sr)
            posb_wait(l, sp)

            @pl.when(l + NIB < SEQ_LEN)
            def _():
                idx_fire(l + NIB, si)

            @pl.when(l + GA < SEQ_LEN)
            def _():
                smi, smr = (j + GA) % NIB, (j + GA) % NRB
                idx_wait(l + GA, smi)
                shift_pass(smi, smr)
                gather_fire(smr)

            @pl.when(l >= NSB)
            def _():
                store_wait(l - NSB, ss)

            rows = row_bufs[sr]
            slab = slab_bufs[ss]
            posb = posb_bufs[sp]

            for g in range(BCH // 16):
                rowv = g * 16 + lanes
                col0 = par_bufs[sr][pl.ds(g * 16, 16)]

                def one_d(d, col):
                    x = plsc.load_gather(rows, [rowv, col])
                    slab[d, pl.ds(g * 16, 16)] = (
                        x + posb[pl.ds(d * 16, 16)])
                    return col + 1

                lax.fori_loop(0, DIM, one_d, col0, unroll=4)

            store_fire(l, ss)

            @pl.when(l + NPB < SEQ_LEN)
            def _():
                posb_fire(l + NPB, sp)
        return carry

    lax.fori_loop(0, SEQ_LEN // RING, one_round, 0)

    for m in range(NSB):
        store_wait(SEQ_LEN - NSB + m, (SEQ_LEN - NSB + m) % NSB)


def kernel(inputs, token_table, pos_table):
    batch, seq_len = inputs.shape
    vocab = token_table.shape[0]
    assert seq_len == SEQ_LEN and batch == NW * BCH and vocab % 2 == 0
    idx_t = inputs.astype(jnp.int32).T
    tab2 = token_table.reshape(vocab // 2, RW)
    posb = jnp.broadcast_to(pos_table[:, :, None],
                            (SEQ_LEN, DIM, 16)).reshape(SEQ_LEN, DIM * 16)

    mesh = plsc.VectorSubcoreMesh(
        core_axis_name="c", subcore_axis_name="s",
        num_cores=NC, num_subcores=NS)

    run = pl.kernel(
        _body,
        out_type=jax.ShapeDtypeStruct((SEQ_LEN, DIM, batch), jnp.float32),
        mesh=mesh,
        scratch_types=[
            [pltpu.VMEM((BCH,), jnp.int32) for _ in range(NIB)],
            [pltpu.VMEM((BCH, RW), jnp.float32) for _ in range(NRB)],
            [pltpu.VMEM((BCH,), jnp.int32) for _ in range(NRB)],
            [pltpu.VMEM((BCH,), jnp.int32) for _ in range(NRB)],
            [pltpu.VMEM((DIM, BCH), jnp.float32) for _ in range(NSB)],
            [pltpu.VMEM((DIM * 16,), jnp.float32) for _ in range(NPB)],
            [pltpu.SemaphoreType.DMA for _ in range(NIB)],
            [pltpu.SemaphoreType.DMA for _ in range(NRB)],
            [pltpu.SemaphoreType.DMA for _ in range(NSB)],
            [pltpu.SemaphoreType.DMA for _ in range(NPB)],
        ],
        compiler_params=pltpu.CompilerParams(
            use_tc_tiling_on_sc=True, needs_layout_passes=False),
    )
    out_phys = run(idx_t, tab2, posb)
    return out_phys.transpose(2, 0, 1)


# speedup vs baseline: 1.0381x; 1.0381x over previous
import jax
import jax.numpy as jnp
from jax import lax
from jax.experimental import pallas as pl
from jax.experimental.pallas import tpu as pltpu
from jax.experimental.pallas import tpu_sc as plsc

SEQ_LEN = 200
DIM = 64
NC = 2
NS = 16
NW = NC * NS
BCH = 128
RW = 128
NST = 4
NIB = 8
NRB = 4
GA = 3
NSB = 2
NPB = 4
RING = 8


def _body(idx_hbm, tab_hbm, posb_hbm, out_hbm,
          idx_bufs, row_bufs, slab_bufs, posb_bufs,
          isems, gsems, ssems, psems):
    wid = lax.axis_index("s") * NC + lax.axis_index("c")
    lanes = lax.iota(jnp.int32, 16)
    zeros16 = lanes - lanes

    def idx_fire(l, s):
        pltpu.async_copy(idx_hbm.at[l, pl.ds(wid * BCH, BCH)],
                         idx_bufs[s], isems[s])

    def idx_wait(l, s):
        pltpu.make_async_copy(idx_hbm.at[l, pl.ds(wid * BCH, BCH)],
                              idx_bufs[s], isems[s]).wait()

    def gather_fire(si, sr):
        n = BCH // NST
        for t in range(NST):
            pltpu.async_copy(
                tab_hbm.at[idx_bufs[si].at[pl.ds(t * n, n)]],
                row_bufs[sr].at[pl.ds(t * n, n)], gsems[sr])

    def gather_wait(si, sr):
        pltpu.make_async_copy(tab_hbm.at[idx_bufs[si]], row_bufs[sr],
                              gsems[sr]).wait()

    def posb_fire(l, s):
        pltpu.async_copy(posb_hbm.at[l], posb_bufs[s], psems[s])

    def posb_wait(l, s):
        pltpu.make_async_copy(posb_hbm.at[l], posb_bufs[s], psems[s]).wait()

    def store_fire(l, s):
        pltpu.async_copy(slab_bufs[s],
                         out_hbm.at[l, :, pl.ds(wid * BCH, BCH)], ssems[s])

    def store_wait(l, s):
        pltpu.make_async_copy(slab_bufs[s],
                              out_hbm.at[l, :, pl.ds(wid * BCH, BCH)],
                              ssems[s]).wait()

    for m in range(NIB):
        idx_fire(m, m)
    for m in range(GA):
        idx_wait(m, m)
        gather_fire(m, m % NRB)
    for m in range(NPB):
        posb_fire(m, m)

    def one_round(r, carry):
        for j in range(RING):
            l = r * RING + j
            si, sr, ss, sp = j % NIB, j % NRB, j % NSB, j % NPB

            gather_wait(si, sr)
            posb_wait(l, sp)

            @pl.when(l + NIB < SEQ_LEN)
            def _():
                idx_fire(l + NIB, si)

            @pl.when(l + GA < SEQ_LEN)
            def _():
                smi = (j + GA) % NIB
                idx_wait(l + GA, smi)
                gather_fire(smi, (j + GA) % NRB)

            @pl.when(l >= NSB)
            def _():
                store_wait(l - NSB, ss)

            rows = row_bufs[sr]
            slab = slab_bufs[ss]
            posb = posb_bufs[sp]

            for g in range(BCH // 16):
                rowv = g * 16 + lanes

                def one_d(d, col):
                    x = plsc.load_gather(rows, [rowv, col])
                    slab[d, pl.ds(g * 16, 16)] = (
                        x + posb[pl.ds(d * 16, 16)])
                    return col + 1

                lax.fori_loop(0, DIM, one_d, zeros16, unroll=4)

            store_fire(l, ss)

            @pl.when(l + NPB < SEQ_LEN)
            def _():
                posb_fire(l + NPB, sp)
        return carry

    lax.fori_loop(0, SEQ_LEN // RING, one_round, 0)

    for m in range(NSB):
        store_wait(SEQ_LEN - NSB + m, (SEQ_LEN - NSB + m) % NSB)


def kernel(inputs, token_table, pos_table):
    batch, seq_len = inputs.shape
    vocab = token_table.shape[0]
    assert seq_len == SEQ_LEN and batch == NW * BCH
    idx_t = inputs.astype(jnp.int32).T
    tab_pad = jnp.pad(token_table, ((0, 0), (0, RW - DIM)))
    posb = jnp.broadcast_to(pos_table[:, :, None],
                            (SEQ_LEN, DIM, 16)).reshape(SEQ_LEN, DIM * 16)

    mesh = plsc.VectorSubcoreMesh(
        core_axis_name="c", subcore_axis_name="s",
        num_cores=NC, num_subcores=NS)

    run = pl.kernel(
        _body,
        out_type=jax.ShapeDtypeStruct((SEQ_LEN, DIM, batch), jnp.float32),
        mesh=mesh,
        scratch_types=[
            [pltpu.VMEM((BCH,), jnp.int32) for _ in range(NIB)],
            [pltpu.VMEM((BCH, RW), jnp.float32) for _ in range(NRB)],
            [pltpu.VMEM((DIM, BCH), jnp.float32) for _ in range(NSB)],
            [pltpu.VMEM((DIM * 16,), jnp.float32) for _ in range(NPB)],
            [pltpu.SemaphoreType.DMA for _ in range(NIB)],
            [pltpu.SemaphoreType.DMA for _ in range(NRB)],
            [pltpu.SemaphoreType.DMA for _ in range(NSB)],
            [pltpu.SemaphoreType.DMA for _ in range(NPB)],
        ],
        compiler_params=pltpu.CompilerParams(
            use_tc_tiling_on_sc=True, needs_layout_passes=False),
    )
    out_phys = run(idx_t, tab_pad, posb)
    return out_phys.transpose(2, 0, 1)


# speedup vs baseline: 1.0520x; 1.0134x over previous
import jax
import jax.numpy as jnp
from jax import lax
from jax.experimental import pallas as pl
from jax.experimental.pallas import tpu as pltpu
from jax.experimental.pallas import tpu_sc as plsc

SEQ_LEN = 200
DIM = 64
NC = 2
NS = 16
NW = NC * NS
BCH = 128
RW = 128
NST = 4
NIB = 8
NRB = 4
GA = 3
NSB = 2
NPB = 4
RING = 8


def _body(idx_hbm, tab_hbm, posb_hbm, out_hbm,
          idx_bufs, row_bufs, slab_bufs, posb_bufs,
          isems, gsems, ssems, psems):
    wid = lax.axis_index("s") * NC + lax.axis_index("c")
    lanes = lax.iota(jnp.int32, 16)
    zeros16 = lanes - lanes

    def idx_fire(l, s):
        pltpu.async_copy(idx_hbm.at[l, pl.ds(wid * BCH, BCH)],
                         idx_bufs[s], isems[s])

    def idx_wait(l, s):
        pltpu.make_async_copy(idx_hbm.at[l, pl.ds(wid * BCH, BCH)],
                              idx_bufs[s], isems[s]).wait()

    def gather_fire(si, sr):
        n = BCH // NST
        for t in range(NST):
            pltpu.async_copy(
                tab_hbm.at[idx_bufs[si].at[pl.ds(t * n, n)]],
                row_bufs[sr].at[pl.ds(t * n, n)], gsems[sr])

    def gather_wait(si, sr):
        pltpu.make_async_copy(tab_hbm.at[idx_bufs[si]], row_bufs[sr],
                              gsems[sr]).wait()

    def posb_fire(l, s):
        pltpu.async_copy(posb_hbm.at[l], posb_bufs[s], psems[s])

    def posb_wait(l, s):
        pltpu.make_async_copy(posb_hbm.at[l], posb_bufs[s], psems[s]).wait()

    def store_fire(l, s):
        pltpu.async_copy(slab_bufs[s],
                         out_hbm.at[l, :, pl.ds(wid * BCH, BCH)], ssems[s])

    def store_wait(l, s):
        pltpu.make_async_copy(slab_bufs[s],
                              out_hbm.at[l, :, pl.ds(wid * BCH, BCH)],
                              ssems[s]).wait()

    for m in range(NIB):
        idx_fire(m, m)
    for m in range(GA):
        idx_wait(m, m)
        gather_fire(m, m % NRB)
    for m in range(NPB):
        posb_fire(m, m)

    def one_round(r, carry):
        for j in range(RING):
            l = r * RING + j
            si, sr, ss, sp = j % NIB, j % NRB, j % NSB, j % NPB

            gather_wait(si, sr)
            posb_wait(l, sp)

            @pl.when(l + NIB < SEQ_LEN)
            def _():
                idx_fire(l + NIB, si)

            @pl.when(l + GA < SEQ_LEN)
            def _():
                smi = (j + GA) % NIB
                idx_wait(l + GA, smi)
                gather_fire(smi, (j + GA) % NRB)

            @pl.when(l >= NSB)
            def _():
                store_wait(l - NSB, ss)

            rows = row_bufs[sr]
            slab = slab_bufs[ss]
            posb = posb_bufs[sp]

            rowvs = tuple(g * 16 + lanes for g in range(BCH // 16))

            def one_d(d, carry):
                col = lax.broadcast_in_dim(d, (16,), ()).astype(jnp.int32)
                pv = posb[pl.ds(d * 16, 16)]
                for g in range(BCH // 16):
                    x = plsc.load_gather(rows, [carry[g], col])
                    slab[d, pl.ds(g * 16, 16)] = x + pv
                return carry

            lax.fori_loop(0, DIM, one_d, rowvs, unroll=4)

            store_fire(l, ss)

            @pl.when(l + NPB < SEQ_LEN)
            def _():
                posb_fire(l + NPB, sp)
        return carry

    lax.fori_loop(0, SEQ_LEN // RING, one_round, 0)

    for m in range(NSB):
        store_wait(SEQ_LEN - NSB + m, (SEQ_LEN - NSB + m) % NSB)


def kernel(inputs, token_table, pos_table):
    batch, seq_len = inputs.shape
    vocab = token_table.shape[0]
    assert seq_len == SEQ_LEN and batch == NW * BCH
    idx_t = inputs.astype(jnp.int32).T
    tab_pad = jnp.pad(token_table, ((0, 0), (0, RW - DIM)))
    posb = jnp.broadcast_to(pos_table[:, :, None],
                            (SEQ_LEN, DIM, 16)).reshape(SEQ_LEN, DIM * 16)

    mesh = plsc.VectorSubcoreMesh(
        core_axis_name="c", subcore_axis_name="s",
        num_cores=NC, num_subcores=NS)

    run = pl.kernel(
        _body,
        out_type=jax.ShapeDtypeStruct((SEQ_LEN, DIM, batch), jnp.float32),
        mesh=mesh,
        scratch_types=[
            [pltpu.VMEM((BCH,), jnp.int32) for _ in range(NIB)],
            [pltpu.VMEM((BCH, RW), jnp.float32) for _ in range(NRB)],
            [pltpu.VMEM((DIM, BCH), jnp.float32) for _ in range(NSB)],
            [pltpu.VMEM((DIM * 16,), jnp.float32) for _ in range(NPB)],
            [pltpu.SemaphoreType.DMA for _ in range(NIB)],
            [pltpu.SemaphoreType.DMA for _ in range(NRB)],
            [pltpu.SemaphoreType.DMA for _ in range(NSB)],
            [pltpu.SemaphoreType.DMA for _ in range(NPB)],
        ],
        compiler_params=pltpu.CompilerParams(
            use_tc_tiling_on_sc=True, needs_layout_passes=False),
    )
    out_phys = run(idx_t, tab_pad, posb)
    return out_phys.transpose(2, 0, 1)
